# baseline (speedup 1.0000x reference)
import functools

import jax
import jax.numpy as jnp
from jax import lax
from jax.experimental import pallas as pl
from jax.experimental.pallas import tpu as pltpu
from jax.experimental.pallas import tpu_sc as plsc

N = 10000
F = 128
E = 320000
H1, C1 = 2, 32
H2, C2 = 1, 32
FFN_OUT = F * 64 * 32

NC, NS, L = 2, 16, 16
NW = NC * NS
EW = E // NW
CH = 80
GPC = CH // L
NGROUP = EW // L
NCHUNK = EW // CH
RI = 4
ZR = 80
NZC = N // ZR


@functools.lru_cache(maxsize=None)
def _make_edge_kernel(H, C):
    HC = H * C
    mesh = plsc.VectorSubcoreMesh(
        core_axis_name="c", subcore_axis_name="s", num_cores=NC, num_subcores=NS
    )

    W = HC + 16

    @functools.partial(
        pl.kernel,
        out_type=jax.ShapeDtypeStruct((NC, N, W), jnp.float32),
        mesh=mesh,
        compiler_params=pltpu.CompilerParams(
            needs_layout_passes=False, use_tc_tiling_on_sc=False),
        scratch_types=[
            pltpu.VMEM((RI, CH), jnp.int32),
            pltpu.VMEM((RI, CH), jnp.int32),
            pltpu.VMEM((2 * CH, HC), jnp.float32),
            pltpu.VMEM((2 * CH, HC), jnp.float32),
            pltpu.VMEM((2 * CH, HC + 16), jnp.float32),
            pltpu.VMEM((HC + 8,), jnp.float32),
            pltpu.VMEM((8 + 16 * H * GPC,), jnp.float32),
            pltpu.VMEM_SHARED((N, HC + 16), jnp.float32),
            pltpu.SemaphoreType.DMA,
            pltpu.SemaphoreType.DMA,
            pltpu.SemaphoreType.DMA,
            pltpu.SemaphoreType.DMA,
            pltpu.SemaphoreType.DMA,
            pltpu.SemaphoreType.DMA,
        ],
    )
    def edge_kernel(src_hbm, dst_hbm, xl_hbm, xr_hbm, att_hbm,
                    num_out,
                    idx_src, idx_dst, xl_rows, xr_rows, scaled, att_v, exbuf,
                    num_acc, isem0, isem1, rsem0, rsem1, ssem0, ssem1):
        cid = lax.axis_index("c")
        sid = lax.axis_index("s")
        wid = cid * NS + sid
        ebase0 = wid * EW
        isems = [isem0, isem1]
        rsems = [rsem0, rsem1]
        ssems = [ssem0, ssem1]

        zero16 = jnp.zeros((L,), jnp.float32)

        @pl.loop(0, 2 * CH)
        def _zero_bufs(r):
            for j in range((HC + 16) // L):
                scaled[r, pl.ds(j * L, L)] = zero16

        @pl.loop(sid, NZC, step=NS)
        def _zero_acc(zc):
            pltpu.sync_copy(scaled.at[pl.ds(0, ZR)], num_acc.at[pl.ds(zc * ZR, ZR)])

        pltpu.sync_copy(att_hbm, att_v.at[pl.ds(8, HC)])
        plsc.subcore_barrier()

        def fire_idx(k, sem):
            base = ebase0 + k * CH
            slot = k % RI
            pltpu.async_copy(src_hbm.at[pl.ds(base, CH)], idx_src.at[slot], sem)
            pltpu.async_copy(dst_hbm.at[pl.ds(base, CH)], idx_dst.at[slot], sem)

        def drain_idx(sem):
            pltpu.make_async_copy(src_hbm.at[pl.ds(0, CH)], idx_src.at[0], sem).wait()
            pltpu.make_async_copy(dst_hbm.at[pl.ds(0, CH)], idx_dst.at[0], sem).wait()

        def fire_rows(k, sem):
            slot = k % RI
            rb = (k % 2) * CH
            pltpu.async_copy(xl_hbm.at[idx_src.at[slot]], xl_rows.at[pl.ds(rb, CH)], sem)
            pltpu.async_copy(xr_hbm.at[idx_dst.at[slot]], xr_rows.at[pl.ds(rb, CH)], sem)

        def drain_rows(sem):
            pltpu.make_async_copy(xl_hbm.at[pl.ds(0, CH)], xl_rows.at[pl.ds(0, CH)], sem).wait()
            pltpu.make_async_copy(xr_hbm.at[pl.ds(0, CH)], xr_rows.at[pl.ds(0, CH)], sem).wait()

        def fire_scat(k, sem):
            slot = k % RI
            rb = (k % 2) * CH
            pltpu.async_copy(scaled.at[pl.ds(rb, CH)], num_acc.at[idx_dst.at[slot]],
                             sem, add=True)

        def drain_scat(sem):
            pltpu.make_async_copy(num_out.at[cid, pl.ds(0, CH)],
                                  scaled.at[pl.ds(0, CH)], sem).wait()

        pltpu.sync_copy(src_hbm.at[pl.ds(ebase0, CH)], idx_src.at[0])
        pltpu.sync_copy(dst_hbm.at[pl.ds(ebase0, CH)], idx_dst.at[0])
        fire_rows(0, rsems[0])
        fire_idx(1, isems[1])

        @pl.loop(0, NCHUNK)
        def _main(k):
            kp = k % 2

            def on_par(p, fn):
                @pl.when(p == 0)
                def _():
                    fn(0)

                @pl.when(p == 1)
                def _():
                    fn(1)

            @pl.when(k >= 2)
            def _():
                on_par(kp, lambda p: drain_scat(ssems[p]))
            @pl.when(k + 1 < NCHUNK)
            def _():
                on_par((k + 1) % 2, lambda p: drain_idx(isems[p]))
                @pl.when(k + 2 < NCHUNK)
                def _():
                    on_par(kp, lambda p: fire_idx(k + 2, isems[p]))
                on_par((k + 1) % 2, lambda p: fire_rows(k + 1, rsems[p]))
            on_par(kp, lambda p: drain_rows(rsems[p]))
            @pl.when(k >= 1)
            def _():
                on_par((k + 1) % 2, lambda p: fire_scat(k - 1, ssems[p]))

            iot = lax.iota(jnp.int32, L)
            att_regs = [att_v[pl.ds(8 + sub * L, L)] for sub in range(HC // L)]
            dmasks = [
                (iot == h).astype(jnp.float32) for h in range(H)
            ]
            for gl in range(GPC):
                eids = kp * CH + gl * L + iot
                P = 2
                parts = [[jnp.zeros((L,), jnp.float32) for _ in range(P)]
                         for _ in range(H)]
                for cg in range(HC):
                    sub = cg // L
                    rot = (cg % L + iot) & (L - 1)
                    ch = rot + sub * L
                    xlc = plsc.load_gather(xl_rows, [eids, ch])
                    xrc = plsc.load_gather(xr_rows, [eids, ch])
                    sv = xlc + xrc
                    ev = jnp.maximum(sv, sv * 0.2)
                    attc = lax.gather(
                        att_regs[sub], rot[:, None],
                        lax.GatherDimensionNumbers(
                            offset_dims=(), collapsed_slice_dims=(0,),
                            start_index_map=(0,)),
                        (1,), mode=lax.GatherScatterMode.PROMISE_IN_BOUNDS)
                    parts[cg // C][cg % P] = parts[cg // C][cg % P] + ev * attc
                exs = [jnp.exp(pp[0] + pp[1]) for pp in parts]
                eb = 8 + 16 * H * gl
                for h in range(H):
                    exbuf[pl.ds(eb + 16 * h, L)] = exs[h]
                base_row = kp * CH + gl * L
                for e in range(L):
                    row = base_row + e
                    splats = [
                        plsc.load_gather(exbuf, [jnp.full((L,), eb + 16 * h + e,
                                                          jnp.int32)])
                        for h in range(H)
                    ]
                    denrow = splats[0] * dmasks[0]
                    for h in range(1, H):
                        denrow = denrow + splats[h] * dmasks[h]
                    for j in range(HC // L):
                        v = xl_rows[row, pl.ds(j * L, L)]
                        scaled[row, pl.ds(j * L, L)] = v * splats[j * L // C]
                    scaled[row, pl.ds(HC, L)] = denrow

        drain_scat(ssems[(NCHUNK - 2) % 2])
        lastslot = (NCHUNK - 1) % RI
        lastrb = ((NCHUNK - 1) % 2) * CH
        pltpu.sync_copy(scaled.at[pl.ds(lastrb, CH)],
                        num_acc.at[idx_dst.at[lastslot]], add=True)
        plsc.subcore_barrier()

        @pl.loop(sid, NZC, step=NS)
        def _copy_out(zc):
            sl = pl.ds(zc * ZR, ZR)
            pltpu.sync_copy(num_acc.at[sl], num_out.at[cid, sl])

    return edge_kernel


def _self_terms(xl, xr, att_flat, H, C):
    s = xl + xr
    e = jnp.maximum(s, 0.2 * s)
    w = e * att_flat
    parts, dens = [], []
    for h in range(H):
        a = jnp.sum(w[:, h * C:(h + 1) * C], axis=1, keepdims=True)
        ex = jnp.exp(a)
        parts.append(xl[:, h * C:(h + 1) * C] * ex)
        dens.append(ex)
    return jnp.concatenate(parts, axis=1), jnp.concatenate(dens, axis=1)


def _proj_body(x_ref, wl_ref, bl_ref, wr_ref, br_ref, att_ref,
               xl_ref, xr_ref, snum_ref, sden_ref, *, H, C):
    x = x_ref[...]
    xl = jnp.dot(x, wl_ref[...], preferred_element_type=jnp.float32) + bl_ref[...]
    xr = jnp.dot(x, wr_ref[...], preferred_element_type=jnp.float32) + br_ref[...]
    snum, sden = _self_terms(xl, xr, att_ref[...], H, C)
    xl_ref[...] = xl
    xr_ref[...] = xr
    snum_ref[...] = snum
    sden_ref[...] = sden


def _combine(acc, snum, sden, bias, bng, bnb, H, C):
    HC = H * C
    num = acc[0, :, 0:HC] + acc[1, :, 0:HC] + snum
    den = acc[0, :, HC:HC + H] + acc[1, :, HC:HC + H] + sden
    denb = jnp.concatenate(
        [jnp.broadcast_to(den[:, h:h + 1], (N, C)) for h in range(H)], axis=1)
    y = num / (denb + 1e-16) + bias
    m = jnp.mean(y, axis=0, keepdims=True)
    v = jnp.mean((y - m) ** 2, axis=0, keepdims=True)
    ybn = (y - m) / jnp.sqrt(v + 1e-5) * bng + bnb
    return jnp.where(ybn > 0, ybn, jnp.exp(ybn) - 1.0)


def _mid_body(acc_ref, snum_ref, sden_ref, bias_ref, bng_ref, bnb_ref,
              x_ref, wl_ref, bl_ref, wr_ref, br_ref, att_ref,
              xl2_ref, xr2_ref, snum2_ref, sden2_ref):
    yact = _combine(acc_ref[...], snum_ref[...], sden_ref[...],
                    bias_ref[...], bng_ref[...], bnb_ref[...], H1, C1)
    ycat = jnp.concatenate([yact, x_ref[...]], axis=1)
    xl2 = jnp.dot(ycat, wl_ref[...], preferred_element_type=jnp.float32) + bl_ref[...]
    xr2 = jnp.dot(ycat, wr_ref[...], preferred_element_type=jnp.float32) + br_ref[...]
    snum2, sden2 = _self_terms(xl2, xr2, att_ref[...], H2, C2)
    xl2_ref[...] = xl2
    xr2_ref[...] = xr2
    snum2_ref[...] = snum2
    sden2_ref[...] = sden2


def _final_body(acc_ref, snum_ref, sden_ref, bias_ref, bng_ref,
                bnb_ref, y_ref):
    y_ref[...] = _combine(acc_ref[...], snum_ref[...],
                          sden_ref[...], bias_ref[...], bng_ref[...],
                          bnb_ref[...], H2, C2)


def _ffn1_body(y_ref, w_ref, b_ref, h_ref):
    k = pl.program_id(0)

    @pl.when(k == 0)
    def _():
        h_ref[...] = b_ref[...]

    h_ref[...] += jnp.dot(y_ref[...], w_ref[...],
                          preferred_element_type=jnp.float32)

    @pl.when(k == pl.num_programs(0) - 1)
    def _():
        h_ref[...] = jnp.maximum(h_ref[...], 0.0)


def _ffn2_body(h_ref, w_ref, b_ref, o_ref):
    o_ref[...] = jnp.maximum(
        jnp.dot(h_ref[...], w_ref[...], preferred_element_type=jnp.float32)
        + b_ref[...], 0.0)


def kernel(input, edge_index, g1_Wl, g1_bl, g1_Wr, g1_br, g1_att, g1_bias,
           bn1_g, bn1_b, g2_Wl, g2_bl, g2_Wr, g2_br, g2_att, g2_bias,
           bn2_g, bn2_b, ffn_W1, ffn_b1, ffn_W2, ffn_b2):
    x = input.reshape(N, F)
    src = edge_index[0]
    dst = edge_index[1]

    HC1 = H1 * C1
    HC2 = H2 * C2

    xl1, xr1, snum1, sden1 = pl.pallas_call(
        functools.partial(_proj_body, H=H1, C=C1),
        out_shape=[
            jax.ShapeDtypeStruct((N, HC1), jnp.float32),
            jax.ShapeDtypeStruct((N, HC1), jnp.float32),
            jax.ShapeDtypeStruct((N, HC1), jnp.float32),
            jax.ShapeDtypeStruct((N, H1), jnp.float32),
        ],
    )(x, g1_Wl, g1_bl.reshape(1, HC1), g1_Wr, g1_br.reshape(1, HC1),
      g1_att.reshape(1, HC1))

    acc1 = _make_edge_kernel(H1, C1)(src, dst, xl1, xr1, g1_att.reshape(HC1))

    xl2, xr2, snum2, sden2 = pl.pallas_call(
        _mid_body,
        compiler_params=pltpu.CompilerParams(vmem_limit_bytes=100 * 2**20),
        out_shape=[
            jax.ShapeDtypeStruct((N, HC2), jnp.float32),
            jax.ShapeDtypeStruct((N, HC2), jnp.float32),
            jax.ShapeDtypeStruct((N, HC2), jnp.float32),
            jax.ShapeDtypeStruct((N, H2), jnp.float32),
        ],
    )(acc1, snum1, sden1, g1_bias.reshape(1, HC1), bn1_g.reshape(1, HC1),
      bn1_b.reshape(1, HC1), x, g2_Wl, g2_bl.reshape(1, HC2), g2_Wr,
      g2_br.reshape(1, HC2), g2_att.reshape(1, HC2))

    acc2 = _make_edge_kernel(H2, C2)(src, dst, xl2, xr2, g2_att.reshape(HC2))

    y2 = pl.pallas_call(
        _final_body,
        out_shape=jax.ShapeDtypeStruct((N, HC2), jnp.float32),
    )(acc2, snum2, sden2, g2_bias.reshape(1, HC2), bn2_g.reshape(1, HC2),
      bn2_b.reshape(1, HC2))

    y_flat = y2.reshape(1, N * HC2)

    KB = 16000
    h = pl.pallas_call(
        _ffn1_body,
        grid=(N * HC2 // KB,),
        in_specs=[
            pl.BlockSpec((1, KB), lambda k: (0, k)),
            pl.BlockSpec((KB, 128), lambda k: (k, 0)),
            pl.BlockSpec((1, 128), lambda k: (0, 0)),
        ],
        out_specs=pl.BlockSpec((1, 128), lambda k: (0, 0)),
        out_shape=jax.ShapeDtypeStruct((1, 128), jnp.float32),
    )(y_flat, ffn_W1, ffn_b1.reshape(1, 128))

    NB = 16384
    out = pl.pallas_call(
        _ffn2_body,
        grid=(FFN_OUT // NB,),
        in_specs=[
            pl.BlockSpec((1, 128), lambda k: (0, 0)),
            pl.BlockSpec((128, NB), lambda k: (0, k)),
            pl.BlockSpec((1, NB), lambda k: (0, k)),
        ],
        out_specs=pl.BlockSpec((1, NB), lambda k: (0, k)),
        out_shape=jax.ShapeDtypeStruct((1, FFN_OUT), jnp.float32),
    )(h, ffn_W2, ffn_b2.reshape(1, FFN_OUT))

    return out.reshape(1, 1, 32, 64, F)


# speedup vs baseline: 1.4880x; 1.4880x over previous
import functools

import jax
import jax.numpy as jnp
from jax import lax
from jax.experimental import pallas as pl
from jax.experimental.pallas import tpu as pltpu
from jax.experimental.pallas import tpu_sc as plsc

N = 10000
F = 128
E = 320000
H1, C1 = 2, 32
H2, C2 = 1, 32
FFN_OUT = F * 64 * 32

NC, NS, L = 2, 16, 16
NW = NC * NS
EW = E // NW
CH = 80
GPC = CH // L
NGROUP = EW // L
NCHUNK = EW // CH
RI = 4
ZR = 80
NZC = N // ZR


@functools.lru_cache(maxsize=None)
def _make_edge_kernel(H, C):
    HC = H * C
    mesh = plsc.VectorSubcoreMesh(
        core_axis_name="c", subcore_axis_name="s", num_cores=NC, num_subcores=NS
    )

    W = HC + 16

    @functools.partial(
        pl.kernel,
        out_type=jax.ShapeDtypeStruct((NC, N, W), jnp.float32),
        mesh=mesh,
        compiler_params=pltpu.CompilerParams(
            needs_layout_passes=False, use_tc_tiling_on_sc=False),
        scratch_types=[
            pltpu.VMEM((RI, CH), jnp.int32),
            pltpu.VMEM((RI, CH), jnp.int32),
            pltpu.VMEM((2 * CH, HC), jnp.float32),
            pltpu.VMEM((2 * CH, HC), jnp.float32),
            pltpu.VMEM((2 * CH, HC + 16), jnp.float32),
            pltpu.VMEM((HC + 8,), jnp.float32),
            pltpu.VMEM((8 + 16 * H * GPC,), jnp.float32),
            pltpu.VMEM_SHARED((N, HC + 16), jnp.float32),
            pltpu.SemaphoreType.DMA,
            pltpu.SemaphoreType.DMA,
            pltpu.SemaphoreType.DMA,
            pltpu.SemaphoreType.DMA,
            pltpu.SemaphoreType.DMA,
            pltpu.SemaphoreType.DMA,
        ],
    )
    def edge_kernel(src_hbm, dst_hbm, xl_hbm, xr_hbm, att_hbm,
                    num_out,
                    idx_src, idx_dst, xl_rows, xr_rows, scaled, att_v, exbuf,
                    num_acc, isem0, isem1, rsem0, rsem1, ssem0, ssem1):
        cid = lax.axis_index("c")
        sid = lax.axis_index("s")
        wid = cid * NS + sid
        ebase0 = wid * EW
        isems = [isem0, isem1]
        rsems = [rsem0, rsem1]
        ssems = [ssem0, ssem1]

        zero16 = jnp.zeros((L,), jnp.float32)

        @pl.loop(0, 2 * CH)
        def _zero_bufs(r):
            for j in range((HC + 16) // L):
                scaled[r, pl.ds(j * L, L)] = zero16

        @pl.loop(sid, NZC, step=NS)
        def _zero_acc(zc):
            pltpu.sync_copy(scaled.at[pl.ds(0, ZR)], num_acc.at[pl.ds(zc * ZR, ZR)])

        pltpu.sync_copy(att_hbm, att_v.at[pl.ds(8, HC)])
        plsc.subcore_barrier()

        def fire_idx(k, sem):
            base = ebase0 + k * CH
            slot = k % RI
            pltpu.async_copy(src_hbm.at[pl.ds(base, CH)], idx_src.at[slot], sem)
            pltpu.async_copy(dst_hbm.at[pl.ds(base, CH)], idx_dst.at[slot], sem)

        def drain_idx(sem):
            pltpu.make_async_copy(src_hbm.at[pl.ds(0, CH)], idx_src.at[0], sem).wait()
            pltpu.make_async_copy(dst_hbm.at[pl.ds(0, CH)], idx_dst.at[0], sem).wait()

        def fire_rows(k, sem):
            slot = k % RI
            rb = (k % 2) * CH
            pltpu.async_copy(xl_hbm.at[idx_src.at[slot]], xl_rows.at[pl.ds(rb, CH)], sem)
            pltpu.async_copy(xr_hbm.at[idx_dst.at[slot]], xr_rows.at[pl.ds(rb, CH)], sem)

        def drain_rows(sem):
            pltpu.make_async_copy(xl_hbm.at[pl.ds(0, CH)], xl_rows.at[pl.ds(0, CH)], sem).wait()
            pltpu.make_async_copy(xr_hbm.at[pl.ds(0, CH)], xr_rows.at[pl.ds(0, CH)], sem).wait()

        def fire_scat(k, sem):
            slot = k % RI
            rb = (k % 2) * CH
            pltpu.async_copy(scaled.at[pl.ds(rb, CH)], num_acc.at[idx_dst.at[slot]],
                             sem, add=True)

        def drain_scat(sem):
            pltpu.make_async_copy(num_out.at[cid, pl.ds(0, CH)],
                                  scaled.at[pl.ds(0, CH)], sem).wait()

        pltpu.sync_copy(src_hbm.at[pl.ds(ebase0, CH)], idx_src.at[0])
        pltpu.sync_copy(dst_hbm.at[pl.ds(ebase0, CH)], idx_dst.at[0])
        fire_rows(0, rsems[0])
        fire_idx(1, isems[1])

        @pl.loop(0, NGROUP)
        def _main(g):
            k = g // GPC
            gl = g % GPC
            kp = k % 2

            def on_par(p, fn):
                @pl.when(p == 0)
                def _():
                    fn(0)

                @pl.when(p == 1)
                def _():
                    fn(1)

            @pl.when(gl == 0)
            def _stage():
                @pl.when(k >= 2)
                def _():
                    on_par(kp, lambda p: drain_scat(ssems[p]))
                @pl.when(k + 1 < NCHUNK)
                def _():
                    on_par((k + 1) % 2, lambda p: drain_idx(isems[p]))
                    @pl.when(k + 2 < NCHUNK)
                    def _():
                        on_par(kp, lambda p: fire_idx(k + 2, isems[p]))
                    on_par((k + 1) % 2, lambda p: fire_rows(k + 1, rsems[p]))
                on_par(kp, lambda p: drain_rows(rsems[p]))
                @pl.when(k >= 1)
                def _():
                    on_par((k + 1) % 2, lambda p: fire_scat(k - 1, ssems[p]))

            iot = lax.iota(jnp.int32, L)
            eids = kp * CH + gl * L + iot
            att_regs = [att_v[pl.ds(8 + sub * L, L)] for sub in range(HC // L)]
            P = 2
            parts = [[jnp.zeros((L,), jnp.float32) for _ in range(P)]
                     for _ in range(H)]
            for cg in range(HC):
                sub = cg // L
                rot = (cg % L + iot) & (L - 1)
                ch = rot + sub * L
                xlc = plsc.load_gather(xl_rows, [eids, ch])
                xrc = plsc.load_gather(xr_rows, [eids, ch])
                sv = xlc + xrc
                ev = jnp.maximum(sv, sv * 0.2)
                attc = lax.gather(
                    att_regs[sub], rot[:, None],
                    lax.GatherDimensionNumbers(
                        offset_dims=(), collapsed_slice_dims=(0,),
                        start_index_map=(0,)),
                    (1,), mode=lax.GatherScatterMode.PROMISE_IN_BOUNDS)
                parts[cg // C][cg % P] = parts[cg // C][cg % P] + ev * attc
            exs = [jnp.exp(pp[0] + pp[1]) for pp in parts]
            for h in range(H):
                exbuf[pl.ds(8 + 16 * h, L)] = exs[h]
            base_row = kp * CH + gl * L
            dmasks = [
                (iot == h).astype(jnp.float32) for h in range(H)
            ]
            for e in range(L):
                row = base_row + e
                splats = [
                    plsc.load_gather(exbuf, [jnp.full((L,), 8 + 16 * h + e,
                                                      jnp.int32)])
                    for h in range(H)
                ]
                denrow = splats[0] * dmasks[0]
                for h in range(1, H):
                    denrow = denrow + splats[h] * dmasks[h]
                for j in range(HC // L):
                    v = xl_rows[row, pl.ds(j * L, L)]
                    scaled[row, pl.ds(j * L, L)] = v * splats[j * L // C]
                scaled[row, pl.ds(HC, L)] = denrow

        drain_scat(ssems[(NCHUNK - 2) % 2])
        lastslot = (NCHUNK - 1) % RI
        lastrb = ((NCHUNK - 1) % 2) * CH
        pltpu.sync_copy(scaled.at[pl.ds(lastrb, CH)],
                        num_acc.at[idx_dst.at[lastslot]], add=True)
        plsc.subcore_barrier()

        @pl.loop(sid, NZC, step=NS)
        def _copy_out(zc):
            sl = pl.ds(zc * ZR, ZR)
            pltpu.sync_copy(num_acc.at[sl], num_out.at[cid, sl])

    return edge_kernel


def _self_terms(xl, xr, att_flat, H, C):
    s = xl + xr
    e = jnp.maximum(s, 0.2 * s)
    w = e * att_flat
    parts, dens = [], []
    for h in range(H):
        a = jnp.sum(w[:, h * C:(h + 1) * C], axis=1, keepdims=True)
        ex = jnp.exp(a)
        parts.append(xl[:, h * C:(h + 1) * C] * ex)
        dens.append(ex)
    return jnp.concatenate(parts, axis=1), jnp.concatenate(dens, axis=1)


def _proj_body(x_ref, wl_ref, bl_ref, wr_ref, br_ref, att_ref,
               xl_ref, xr_ref, snum_ref, sden_ref, *, H, C):
    x = x_ref[...]
    xl = jnp.dot(x, wl_ref[...], preferred_element_type=jnp.float32) + bl_ref[...]
    xr = jnp.dot(x, wr_ref[...], preferred_element_type=jnp.float32) + br_ref[...]
    snum, sden = _self_terms(xl, xr, att_ref[...], H, C)
    xl_ref[...] = xl
    xr_ref[...] = xr
    snum_ref[...] = snum
    sden_ref[...] = sden


def _combine(acc, snum, sden, bias, bng, bnb, H, C):
    HC = H * C
    num = acc[0, :, 0:HC] + acc[1, :, 0:HC] + snum
    den = acc[0, :, HC:HC + H] + acc[1, :, HC:HC + H] + sden
    denb = jnp.concatenate(
        [jnp.broadcast_to(den[:, h:h + 1], (N, C)) for h in range(H)], axis=1)
    y = num / (denb + 1e-16) + bias
    m = jnp.mean(y, axis=0, keepdims=True)
    v = jnp.mean((y - m) ** 2, axis=0, keepdims=True)
    ybn = (y - m) / jnp.sqrt(v + 1e-5) * bng + bnb
    return jnp.where(ybn > 0, ybn, jnp.exp(ybn) - 1.0)


def _mid_body(acc_ref, snum_ref, sden_ref, bias_ref, bng_ref, bnb_ref,
              x_ref, wl_ref, bl_ref, wr_ref, br_ref, att_ref,
              xl2_ref, xr2_ref, snum2_ref, sden2_ref):
    yact = _combine(acc_ref[...], snum_ref[...], sden_ref[...],
                    bias_ref[...], bng_ref[...], bnb_ref[...], H1, C1)
    ycat = jnp.concatenate([yact, x_ref[...]], axis=1)
    xl2 = jnp.dot(ycat, wl_ref[...], preferred_element_type=jnp.float32) + bl_ref[...]
    xr2 = jnp.dot(ycat, wr_ref[...], preferred_element_type=jnp.float32) + br_ref[...]
    snum2, sden2 = _self_terms(xl2, xr2, att_ref[...], H2, C2)
    xl2_ref[...] = xl2
    xr2_ref[...] = xr2
    snum2_ref[...] = snum2
    sden2_ref[...] = sden2


def _final_body(acc_ref, snum_ref, sden_ref, bias_ref, bng_ref,
                bnb_ref, y_ref):
    y_ref[...] = _combine(acc_ref[...], snum_ref[...],
                          sden_ref[...], bias_ref[...], bng_ref[...],
                          bnb_ref[...], H2, C2)


def _ffn1_body(y_ref, w_ref, b_ref, h_ref):
    k = pl.program_id(0)

    @pl.when(k == 0)
    def _():
        h_ref[...] = b_ref[...]

    h_ref[...] += jnp.dot(y_ref[...], w_ref[...],
                          preferred_element_type=jnp.float32)

    @pl.when(k == pl.num_programs(0) - 1)
    def _():
        h_ref[...] = jnp.maximum(h_ref[...], 0.0)


def _ffn2_body(h_ref, w_ref, b_ref, o_ref):
    o_ref[...] = jnp.maximum(
        jnp.dot(h_ref[...], w_ref[...], preferred_element_type=jnp.float32)
        + b_ref[...], 0.0)


def kernel(input, edge_index, g1_Wl, g1_bl, g1_Wr, g1_br, g1_att, g1_bias,
           bn1_g, bn1_b, g2_Wl, g2_bl, g2_Wr, g2_br, g2_att, g2_bias,
           bn2_g, bn2_b, ffn_W1, ffn_b1, ffn_W2, ffn_b2):
    x = input.reshape(N, F)
    src = edge_index[0]
    dst = edge_index[1]

    HC1 = H1 * C1
    HC2 = H2 * C2

    xl1, xr1, snum1, sden1 = pl.pallas_call(
        functools.partial(_proj_body, H=H1, C=C1),
        out_shape=[
            jax.ShapeDtypeStruct((N, HC1), jnp.float32),
            jax.ShapeDtypeStruct((N, HC1), jnp.float32),
            jax.ShapeDtypeStruct((N, HC1), jnp.float32),
            jax.ShapeDtypeStruct((N, H1), jnp.float32),
        ],
    )(x, g1_Wl, g1_bl.reshape(1, HC1), g1_Wr, g1_br.reshape(1, HC1),
      g1_att.reshape(1, HC1))

    acc1 = _make_edge_kernel(H1, C1)(src, dst, xl1, xr1, g1_att.reshape(HC1))

    xl2, xr2, snum2, sden2 = pl.pallas_call(
        _mid_body,
        compiler_params=pltpu.CompilerParams(vmem_limit_bytes=100 * 2**20),
        out_shape=[
            jax.ShapeDtypeStruct((N, HC2), jnp.float32),
            jax.ShapeDtypeStruct((N, HC2), jnp.float32),
            jax.ShapeDtypeStruct((N, HC2), jnp.float32),
            jax.ShapeDtypeStruct((N, H2), jnp.float32),
        ],
    )(acc1, snum1, sden1, g1_bias.reshape(1, HC1), bn1_g.reshape(1, HC1),
      bn1_b.reshape(1, HC1), x, g2_Wl, g2_bl.reshape(1, HC2), g2_Wr,
      g2_br.reshape(1, HC2), g2_att.reshape(1, HC2))

    acc2 = _make_edge_kernel(H2, C2)(src, dst, xl2, xr2, g2_att.reshape(HC2))

    y2 = pl.pallas_call(
        _final_body,
        out_shape=jax.ShapeDtypeStruct((N, HC2), jnp.float32),
    )(acc2, snum2, sden2, g2_bias.reshape(1, HC2), bn2_g.reshape(1, HC2),
      bn2_b.reshape(1, HC2))

    y_flat = y2.reshape(1, N * HC2)

    KB = 16000
    h = pl.pallas_call(
        _ffn1_body,
        grid=(N * HC2 // KB,),
        in_specs=[
            pl.BlockSpec((1, KB), lambda k: (0, k)),
            pl.BlockSpec((KB, 128), lambda k: (k, 0)),
            pl.BlockSpec((1, 128), lambda k: (0, 0)),
        ],
        out_specs=pl.BlockSpec((1, 128), lambda k: (0, 0)),
        out_shape=jax.ShapeDtypeStruct((1, 128), jnp.float32),
    )(y_flat, ffn_W1, ffn_b1.reshape(1, 128))

    NB = 16384
    out = pl.pallas_call(
        _ffn2_body,
        grid=(FFN_OUT // NB,),
        in_specs=[
            pl.BlockSpec((1, 128), lambda k: (0, 0)),
            pl.BlockSpec((128, NB), lambda k: (0, k)),
            pl.BlockSpec((1, NB), lambda k: (0, k)),
        ],
        out_specs=pl.BlockSpec((1, NB), lambda k: (0, k)),
        out_shape=jax.ShapeDtypeStruct((1, FFN_OUT), jnp.float32),
    )(h, ffn_W2, ffn_b2.reshape(1, FFN_OUT))

    return out.reshape(1, 1, 32, 64, F)
